# baseline (speedup 1.0000x reference)
import functools

import jax
import jax.numpy as jnp
from jax import lax
from jax.experimental import pallas as pl
from jax.experimental.pallas import tpu as pltpu
from jax.experimental.pallas import tpu_sc as plsc

D = 32
B = 16384 * 50

NC = 2
NS = 16
NW = NC * NS
B_PER_W = B // NW

G = 128
HALF = 4 * G
N_HALVES = B_PER_W // HALF
N_BLOCKS = N_HALVES // 2

TR = D // 8
TC_ALL = B // G

_mesh = plsc.VectorSubcoreMesh(core_axis_name="c", subcore_axis_name="s")


@functools.partial(
    pl.kernel,
    mesh=_mesh,
    out_type=jax.ShapeDtypeStruct((TR, TC_ALL, 8, G), jnp.float32),
    scratch_types=[
        pltpu.VMEM((8, G), jnp.int32),
        pltpu.VMEM((8, G), jnp.int32),
        pltpu.VMEM((HALF, D), jnp.float32),
        pltpu.VMEM((HALF, D), jnp.float32),
        pltpu.VMEM((TR, 4, 8, G), jnp.float32),
        pltpu.VMEM((TR, 4, 8, G), jnp.float32),
        pltpu.SemaphoreType.DMA,
        pltpu.SemaphoreType.DMA,
        pltpu.SemaphoreType.DMA,
        pltpu.SemaphoreType.DMA,
        pltpu.SemaphoreType.DMA,
        pltpu.SemaphoreType.DMA,
    ],
    compiler_params=pltpu.CompilerParams(
        use_tc_tiling_on_sc=False, needs_layout_passes=False),
)
def _gather_kernel(idx_hbm, table_hbm, out_hbm,
                   idxA, idxB, rows0, rows1, tout0, tout1,
                   isA, isB, gs0, gs1, os0, os1):
    wid = lax.axis_index("s") * NC + lax.axis_index("c")
    idx_row0 = wid * (B_PER_W // G)
    tc0_w = wid * (B_PER_W // G)

    idx_v = (idxA, idxB)
    rows_v = (rows0, rows1)
    tout_v = (tout0, tout1)
    isem = (isA, isB)
    gsem = (gs0, gs1)
    osem = (os0, os1)

    lane_iota = lax.iota(jnp.int32, 16)

    def idx_load(tb, p):
        pltpu.make_async_copy(
            idx_hbm.at[pl.ds(idx_row0 + tb * 8, 8)], idx_v[p], isem[p]).start()

    def idx_wait(p):
        pltpu.make_async_copy(idx_hbm.at[pl.ds(0, 8)], idx_v[p], isem[p]).wait()

    def fire(rb, p, jbase):
        for j in range(4):
            pltpu.make_async_copy(
                table_hbm.at[idx_v[p].at[jbase + j]],
                rows_v[rb].at[pl.ds(j * G, G)], gsem[rb]).start()

    def drain(rb):
        pltpu.make_async_copy(
            table_hbm.at[pl.ds(0, HALF)], rows_v[rb], gsem[rb]).wait()

    def transpose_half(rb):
        def body(tcrel, _):
            row_base = tcrel * G
            for d in range(D):
                col = jnp.full((16,), d, jnp.int32)
                for l0 in range(0, G, 16):
                    rows = row_base + l0 + lane_iota
                    rows = row_base + l0 + lane_iota
                    v = plsc.load_gather(rows_v[rb], [rows, col])
                    tout_v[rb][d // 8, tcrel, d % 8, pl.ds(l0, 16)] = v
            return 0
        lax.fori_loop(0, 4, body, 0)

    def out_start(h, rb):
        pltpu.make_async_copy(
            tout_v[rb],
            out_hbm.at[:, pl.ds(tc0_w + h * 4, 4)],
            osem[rb]).start()

    def out_wait(rb):
        pltpu.make_async_copy(
            tout_v[rb], out_hbm.at[:, pl.ds(0, 4)], osem[rb]).wait()

    def finish_prev(h, rb_prev):
        drain(rb_prev)
        transpose_half(rb_prev)
        out_start(h - 1, rb_prev)

    idx_load(0, 0)
    idx_load(1, 1)
    idx_wait(0)
    fire(0, 0, 0)

    def quad(T, _):
        h = 4 * T + 1

        @pl.when(T > 0)
        def _():
            out_wait(1)
        fire(1, 0, 4)
        finish_prev(h, 0)

        h = 4 * T + 2
        idx_wait(1)
        out_wait(0)
        fire(0, 1, 0)
        finish_prev(h, 1)
        idx_load(2 * T + 2, 0)

        h = 4 * T + 3
        out_wait(1)
        fire(1, 1, 4)
        finish_prev(h, 0)

        h = 4 * T + 4
        idx_wait(0)
        out_wait(0)
        fire(0, 0, 0)
        finish_prev(h, 1)

        @pl.when(T < N_BLOCKS // 2 - 1)
        def _():
            idx_load(2 * T + 3, 1)

        return 0

    lax.fori_loop(0, (N_HALVES - 2) // 4, quad, 0)

    out_wait(1)
    fire(1, 0, 4)
    drain(0)
    transpose_half(0)
    out_start(48, 0)
    drain(1)
    transpose_half(1)
    out_start(49, 1)
    out_wait(0)
    out_wait(1)


def kernel(camera_ids, weight):
    ids = camera_ids.reshape(-1).astype(jnp.int32)
    idx2d = ids.reshape(B // G, G)
    out4d = _gather_kernel(idx2d, weight)
    return out4d.transpose(1, 3, 0, 2).reshape(B, D)


# speedup vs baseline: 1.2959x; 1.2959x over previous
import functools

import jax
import jax.numpy as jnp
from jax import lax
from jax.experimental import pallas as pl
from jax.experimental.pallas import tpu as pltpu
from jax.experimental.pallas import tpu_sc as plsc

D = 32
B = 16384 * 50

NC = 2
NS = 16
NW = NC * NS
B_PER_W = B // NW

G = 128
HALF = 4 * G
N_HALVES = B_PER_W // HALF
N_BLOCKS = N_HALVES // 2

TR = D // 8
TC_ALL = B // G

_mesh = plsc.VectorSubcoreMesh(core_axis_name="c", subcore_axis_name="s")


@functools.partial(
    pl.kernel,
    mesh=_mesh,
    out_type=jax.ShapeDtypeStruct((TR, TC_ALL, 8, G), jnp.float32),
    scratch_types=[
        pltpu.VMEM((8, G), jnp.int32),
        pltpu.VMEM((8, G), jnp.int32),
        pltpu.VMEM((HALF, D), jnp.float32),
        pltpu.VMEM((HALF, D), jnp.float32),
        pltpu.VMEM((TR, 4, 8, G), jnp.float32),
        pltpu.VMEM((TR, 4, 8, G), jnp.float32),
        pltpu.SemaphoreType.DMA,
        pltpu.SemaphoreType.DMA,
        pltpu.SemaphoreType.DMA,
        pltpu.SemaphoreType.DMA,
        pltpu.SemaphoreType.DMA,
        pltpu.SemaphoreType.DMA,
    ],
    compiler_params=pltpu.CompilerParams(
        use_tc_tiling_on_sc=False, needs_layout_passes=False),
)
def _gather_kernel(idx_hbm, table_hbm, out_hbm,
                   idxA, idxB, rows0, rows1, tout0, tout1,
                   isA, isB, gs0, gs1, os0, os1):
    wid = lax.axis_index("s") * NC + lax.axis_index("c")
    idx_row0 = wid * (B_PER_W // G)
    tc0_w = wid * (B_PER_W // G)

    idx_v = (idxA, idxB)
    rows_v = (rows0, rows1)
    tout_v = (tout0, tout1)
    isem = (isA, isB)
    gsem = (gs0, gs1)
    osem = (os0, os1)

    lane_iota = lax.iota(jnp.int32, 16)

    def idx_load(tb, p):
        pltpu.make_async_copy(
            idx_hbm.at[pl.ds(idx_row0 + tb * 8, 8)], idx_v[p], isem[p]).start()

    def idx_wait(p):
        pltpu.make_async_copy(idx_hbm.at[pl.ds(0, 8)], idx_v[p], isem[p]).wait()

    def fire(rb, p, jbase):
        for j in range(4):
            pltpu.make_async_copy(
                table_hbm.at[idx_v[p].at[jbase + j]],
                rows_v[rb].at[pl.ds(j * G, G)], gsem[rb]).start()

    def drain(rb):
        pltpu.make_async_copy(
            table_hbm.at[pl.ds(0, HALF)], rows_v[rb], gsem[rb]).wait()

    def transpose_half(rb):
        def body(tcrel, _):
            row_base = tcrel * G

            for l0 in range(0, G, 16):
                rows = row_base + l0 + lane_iota
                vs = [
                    plsc.load_gather(
                        rows_v[rb], [rows, jnp.full((16,), d, jnp.int32)])
                    for d in range(D)
                ]
                for d in range(D):
                    tout_v[rb][d // 8, tcrel, d % 8, pl.ds(l0, 16)] = vs[d]

            return 0
        lax.fori_loop(0, 4, body, 0)

    def out_start(h, rb):
        pltpu.make_async_copy(
            tout_v[rb],
            out_hbm.at[:, pl.ds(tc0_w + h * 4, 4)],
            osem[rb]).start()

    def out_wait(rb):
        pltpu.make_async_copy(
            tout_v[rb], out_hbm.at[:, pl.ds(0, 4)], osem[rb]).wait()

    def finish_prev(h, rb_prev):
        drain(rb_prev)
        transpose_half(rb_prev)
        out_start(h - 1, rb_prev)

    idx_load(0, 0)
    idx_load(1, 1)
    idx_wait(0)
    fire(0, 0, 0)

    def quad(T, _):
        h = 4 * T + 1

        @pl.when(T > 0)
        def _():
            out_wait(1)
        fire(1, 0, 4)
        finish_prev(h, 0)

        h = 4 * T + 2
        idx_wait(1)
        out_wait(0)
        fire(0, 1, 0)
        finish_prev(h, 1)
        idx_load(2 * T + 2, 0)

        h = 4 * T + 3
        out_wait(1)
        fire(1, 1, 4)
        finish_prev(h, 0)

        h = 4 * T + 4
        idx_wait(0)
        out_wait(0)
        fire(0, 0, 0)
        finish_prev(h, 1)

        @pl.when(T < N_BLOCKS // 2 - 1)
        def _():
            idx_load(2 * T + 3, 1)

        return 0

    lax.fori_loop(0, (N_HALVES - 2) // 4, quad, 0)

    out_wait(1)
    fire(1, 0, 4)
    drain(0)
    transpose_half(0)
    out_start(48, 0)
    drain(1)
    transpose_half(1)
    out_start(49, 1)
    out_wait(0)
    out_wait(1)


def kernel(camera_ids, weight):
    ids = camera_ids.reshape(-1).astype(jnp.int32)
    idx2d = ids.reshape(B // G, G)
    out4d = _gather_kernel(idx2d, weight)
    return out4d.transpose(1, 3, 0, 2).reshape(B, D)


# speedup vs baseline: 1.7808x; 1.3742x over previous
import functools

import jax
import jax.numpy as jnp
from jax import lax
from jax.experimental import pallas as pl
from jax.experimental.pallas import tpu as pltpu
from jax.experimental.pallas import tpu_sc as plsc

D = 32
B = 16384 * 50

NC = 2
NS = 16
NW = NC * NS
B_PER_W = B // NW

G = 128
HALF = 4 * G
N_HALVES = B_PER_W // HALF
N_BLOCKS = N_HALVES // 2

TR = D // 8
TC_ALL = B // G

S2R = 40
S2W = 136

_mesh = plsc.VectorSubcoreMesh(core_axis_name="c", subcore_axis_name="s")


@functools.partial(
    pl.kernel,
    mesh=_mesh,
    out_type=jax.ShapeDtypeStruct((TR * TC_ALL * 8, G), jnp.float32),
    scratch_types=[
        pltpu.VMEM((8, G), jnp.int32),
        pltpu.VMEM((8, G), jnp.int32),
        pltpu.VMEM((HALF, D), jnp.float32),
        pltpu.VMEM((HALF, D), jnp.float32),
        pltpu.VMEM((TR * S2R, S2W), jnp.float32),
        pltpu.VMEM((TR * S2R, S2W), jnp.float32),
        pltpu.SemaphoreType.DMA,
        pltpu.SemaphoreType.DMA,
        pltpu.SemaphoreType.DMA,
        pltpu.SemaphoreType.DMA,
        pltpu.SemaphoreType.DMA,
        pltpu.SemaphoreType.DMA,
    ],
    compiler_params=pltpu.CompilerParams(
        use_tc_tiling_on_sc=False, needs_layout_passes=False),
)
def _gather_kernel(idx_hbm, table_hbm, out_hbm,
                   idxA, idxB, rows0, rows1, st0, st1,
                   isA, isB, gs0, gs1, os0, os1):
    wid = lax.axis_index("s") * NC + lax.axis_index("c")
    idx_row0 = wid * (B_PER_W // G)
    tc0_w = wid * (B_PER_W // G)

    idx_v = (idxA, idxB)
    rows_v = (rows0, rows1)
    st_v = (st0, st1)
    isem = (isA, isB)
    gsem = (gs0, gs1)
    osem = (os0, os1)

    iota16 = lax.iota(jnp.int32, 16)
    srow0 = (iota16 >> 3) * S2R + (iota16 & 7)
    srow1 = srow0 + 2 * S2R
    zeros16 = iota16 * 0

    def idx_load(tb, p):
        pltpu.make_async_copy(
            idx_hbm.at[pl.ds(idx_row0 + tb * 8, 8)], idx_v[p], isem[p]).start()

    def idx_wait(p):
        pltpu.make_async_copy(idx_hbm.at[pl.ds(0, 8)], idx_v[p], isem[p]).wait()

    def fire(rb, p, jbase):
        for j in range(4):
            pltpu.make_async_copy(
                table_hbm.at[idx_v[p].at[jbase + j]],
                rows_v[rb].at[pl.ds(j * G, G)], gsem[rb]).start()

    def drain(rb):
        pltpu.make_async_copy(
            table_hbm.at[pl.ds(0, HALF)], rows_v[rb], gsem[rb]).wait()

    def transpose_half(rb):
        def body(q, _):
            for u in range(4):
                r = q * 4 + u
                tcrel8 = (r >> 7) * 8
                col = r & (G - 1)
                colv = zeros16 + col
                r0v = srow0 + tcrel8
                r1v = srow1 + tcrel8
                v0 = rows_v[rb][r, pl.ds(0, 16)]
                v1 = rows_v[rb][r, pl.ds(16, 16)]
                plsc.store_scatter(st_v[rb], [r0v, colv], v0)
                plsc.store_scatter(st_v[rb], [r1v, colv], v1)
            return 0
        lax.fori_loop(0, HALF // 4, body, 0)

    def out_copies(h, rb, dummy):
        tc = tc0_w + h * 4
        for tr in range(TR):
            c = pltpu.make_async_copy(
                st_v[rb].at[pl.ds(tr * S2R, 32), pl.ds(0, G)],
                out_hbm.at[pl.ds((tr * TC_ALL + tc) * 8, 32)],
                osem[rb])
            if dummy:
                c.wait()
            else:
                c.start()

    def out_start(h, rb):
        out_copies(h, rb, False)

    def out_wait(rb):
        out_copies(0, rb, True)

    def finish_prev(h, rb_prev):
        drain(rb_prev)
        transpose_half(rb_prev)
        out_start(h - 1, rb_prev)

    idx_load(0, 0)
    idx_load(1, 1)
    idx_wait(0)
    fire(0, 0, 0)

    def quad(T, _):
        h = 4 * T + 1

        @pl.when(T > 0)
        def _():
            out_wait(1)
        fire(1, 0, 4)
        finish_prev(h, 0)

        h = 4 * T + 2
        idx_wait(1)
        out_wait(0)
        fire(0, 1, 0)
        finish_prev(h, 1)
        idx_load(2 * T + 2, 0)

        h = 4 * T + 3
        out_wait(1)
        fire(1, 1, 4)
        finish_prev(h, 0)

        h = 4 * T + 4
        idx_wait(0)
        out_wait(0)
        fire(0, 0, 0)
        finish_prev(h, 1)

        @pl.when(T < N_BLOCKS // 2 - 1)
        def _():
            idx_load(2 * T + 3, 1)

        return 0

    lax.fori_loop(0, (N_HALVES - 2) // 4, quad, 0)

    out_wait(1)
    fire(1, 0, 4)
    drain(0)
    transpose_half(0)
    out_start(48, 0)
    drain(1)
    transpose_half(1)
    out_start(49, 1)
    out_wait(0)
    out_wait(1)


def kernel(camera_ids, weight):
    ids = camera_ids.reshape(-1).astype(jnp.int32)
    idx2d = ids.reshape(B // G, G)
    out2d = _gather_kernel(idx2d, weight)
    out4d = out2d.reshape(TR, TC_ALL, 8, G)
    return out4d.transpose(1, 3, 0, 2).reshape(B, D)
